# baseline (speedup 1.0000x reference)
import functools

import jax
import jax.numpy as jnp
from jax import lax
from jax.experimental import pallas as pl
from jax.experimental.pallas import tpu as pltpu
from jax.experimental.pallas import tpu_sc as plsc

CAP = 8192
G = 8
HD = 128
NEW = 32
BLK = 512
NBLK = CAP // BLK

NC, NS = 2, 16
NW = NC * NS
RPW = CAP // NW
CHR = 32
NCH = RPW // CHR


def _keys_body(cl_ref, keys, new_keys, out_k):
    i = pl.program_id(0)
    blk_start = i * BLK
    out_k[...] = keys[...]

    cl = cl_ref[0]

    @pl.when(jnp.logical_and(cl + NEW > blk_start, cl < blk_start + BLK))
    def _():
        def body(r, carry):
            dest = cl + r - blk_start

            @pl.when(jnp.logical_and(dest >= 0, dest < BLK))
            def _():
                out_k[pl.ds(dest, 1)] = new_keys[pl.ds(r, 1)]

            return carry

        lax.fori_loop(0, NEW, body, 0)


_sc_mesh = plsc.VectorSubcoreMesh(
    core_axis_name="c", subcore_axis_name="s", num_cores=NC, num_subcores=NS
)


@functools.partial(
    pl.kernel,
    out_type=jax.ShapeDtypeStruct((CAP, G, HD), jnp.float32),
    mesh=_sc_mesh,
    scratch_types=[
        pltpu.VMEM((2, CHR, G, HD), jnp.float32),
        pltpu.VMEM((16,), jnp.int32),
        pltpu.SemaphoreType.DMA((2,)),
        pltpu.SemaphoreType.DMA((2,)),
    ],
    compiler_params=pltpu.CompilerParams(use_tc_tiling_on_sc=True),
)
def _values_sc(values, cl_vec, new_values, out_v, bufs, cl_vmem,
               in_sem, out_sem):
    wid = lax.axis_index("c") * NS + lax.axis_index("s")
    base = wid * RPW

    pltpu.sync_copy(cl_vec, cl_vmem)
    cl = cl_vmem[...][0]

    in_d = [None] * NCH
    out_d = [None] * NCH
    for step in range(NCH + 1):
        c = step
        if c < NCH:
            b = c % 2
            if c >= 2:
                out_d[c - 2].wait()
            sl = pl.ds(base + c * CHR, CHR)
            in_d[c] = pltpu.make_async_copy(values.at[sl], bufs.at[b],
                                            in_sem.at[b])
            in_d[c].start()
        co = step - 1
        if co >= 0:
            b = co % 2
            in_d[co].wait()
            sl = pl.ds(base + co * CHR, CHR)
            out_d[co] = pltpu.make_async_copy(bufs.at[b], out_v.at[sl],
                                              out_sem.at[b])
            out_d[co].start()
    out_d[NCH - 2].wait()
    out_d[NCH - 1].wait()

    def body(r, carry):
        dest = cl + r

        @pl.when(jnp.logical_and(dest >= base, dest < base + RPW))
        def _():
            pltpu.sync_copy(new_values.at[pl.ds(r, 1)],
                            out_v.at[pl.ds(dest, 1)])

        return carry

    lax.fori_loop(0, NEW, body, 0)


def kernel(keys, values, current_length, new_keys, new_values):
    cl1 = current_length.reshape(1)
    out_k = pl.pallas_call(
        _keys_body,
        grid=(NBLK,),
        in_specs=[
            pl.BlockSpec(memory_space=pltpu.SMEM),
            pl.BlockSpec((BLK, G, HD), lambda i: (i, 0, 0)),
            pl.BlockSpec((NEW, G, HD), lambda i: (0, 0, 0)),
        ],
        out_specs=pl.BlockSpec((BLK, G, HD), lambda i: (i, 0, 0)),
        out_shape=jax.ShapeDtypeStruct((CAP, G, HD), jnp.float32),
        compiler_params=pltpu.CompilerParams(
            dimension_semantics=("arbitrary",),
        ),
    )(cl1, keys, new_keys)
    cl_vec = jnp.full((16,), current_length, dtype=jnp.int32)
    out_v = _values_sc(values, cl_vec, new_values)
    return (out_k, out_v, current_length + NEW)


# speedup vs baseline: 1.0070x; 1.0070x over previous
import functools

import jax
import jax.numpy as jnp
from jax import lax
from jax.experimental import pallas as pl
from jax.experimental.pallas import tpu as pltpu
from jax.experimental.pallas import tpu_sc as plsc

CAP = 8192
G = 8
HD = 128
NEW = 32
BLK = 512
NBLK = CAP // BLK

NC, NS = 2, 16
NW = NC * NS
RPW = CAP // NW
CHR = 32
NCH = RPW // CHR


def _keys_body(cl_ref, keys, new_keys, out_k):
    i = pl.program_id(0)
    blk_start = i * BLK
    out_k[...] = keys[...]

    cl = cl_ref[0]

    @pl.when(jnp.logical_and(cl + NEW > blk_start, cl < blk_start + BLK))
    def _():
        def body(r, carry):
            dest = cl + r - blk_start

            @pl.when(jnp.logical_and(dest >= 0, dest < BLK))
            def _():
                out_k[pl.ds(dest, 1)] = new_keys[pl.ds(r, 1)]

            return carry

        lax.fori_loop(0, NEW, body, 0)


_sc_mesh = plsc.VectorSubcoreMesh(
    core_axis_name="c", subcore_axis_name="s", num_cores=NC, num_subcores=NS
)


@functools.partial(
    pl.kernel,
    out_type=jax.ShapeDtypeStruct((CAP, G, HD), jnp.float32),
    mesh=_sc_mesh,
    scratch_types=[
        pltpu.VMEM((2, CHR, G, HD), jnp.float32),
        pltpu.VMEM((16,), jnp.int32),
        pltpu.SemaphoreType.DMA((2,)),
        pltpu.SemaphoreType.DMA((2,)),
    ],
    compiler_params=pltpu.CompilerParams(use_tc_tiling_on_sc=True),
)
def _values_sc(values, cl_vec, new_values, out_v, bufs, cl_vmem,
               in_sem, out_sem):
    wid = lax.axis_index("c") * NS + lax.axis_index("s")
    base = wid * RPW

    pltpu.sync_copy(cl_vec, cl_vmem)
    cl = cl_vmem[...][0]

    in_d = [None] * NCH
    out_d = [None] * NCH
    for step in range(NCH + 1):
        c = step
        if c < NCH:
            b = c % 2
            if c >= 2:
                out_d[c - 2].wait()
            sl = pl.ds(base + c * CHR, CHR)
            in_d[c] = pltpu.make_async_copy(values.at[sl], bufs.at[b],
                                            in_sem.at[b])
            in_d[c].start()
        co = step - 1
        if co >= 0:
            b = co % 2
            in_d[co].wait()
            sl = pl.ds(base + co * CHR, CHR)
            out_d[co] = pltpu.make_async_copy(bufs.at[b], out_v.at[sl],
                                              out_sem.at[b])
            out_d[co].start()
    out_d[NCH - 2].wait()
    out_d[NCH - 1].wait()

    def body(r, carry):
        dest = cl + r

        @pl.when(jnp.logical_and(dest >= base, dest < base + RPW))
        def _():
            pltpu.sync_copy(new_values.at[pl.ds(r, 1)],
                            out_v.at[pl.ds(dest, 1)])

        return carry

    lax.fori_loop(0, NEW, body, 0)


def kernel(keys, values, current_length, new_keys, new_values):
    cl1 = current_length.reshape(1)
    cl_vec = jnp.full((16,), current_length, dtype=jnp.int32)
    out_v = _values_sc(values, cl_vec, new_values)
    out_k = pl.pallas_call(
        _keys_body,
        grid=(NBLK,),
        in_specs=[
            pl.BlockSpec(memory_space=pltpu.SMEM),
            pl.BlockSpec((BLK, G, HD), lambda i: (i, 0, 0)),
            pl.BlockSpec((NEW, G, HD), lambda i: (0, 0, 0)),
        ],
        out_specs=pl.BlockSpec((BLK, G, HD), lambda i: (i, 0, 0)),
        out_shape=jax.ShapeDtypeStruct((CAP, G, HD), jnp.float32),
        compiler_params=pltpu.CompilerParams(
            dimension_semantics=("arbitrary",),
        ),
    )(cl1, keys, new_keys)
    return (out_k, out_v, current_length + NEW)


# speedup vs baseline: 2.2475x; 2.2319x over previous
import jax
import jax.numpy as jnp
from jax.experimental import pallas as pl
from jax.experimental.pallas import tpu as pltpu

CAP = 8192
G = 8
HD = 128
NEW = 32
BLK = 1024
NBLK = CAP // BLK


def _extend_body(cl_ref, keys, values, new_keys, new_values, out_k, out_v):
    i = pl.program_id(0)
    blk_start = i * BLK
    out_k[...] = keys[...]
    out_v[...] = values[...]

    cl = cl_ref[0]

    @pl.when(jnp.logical_and(cl + NEW > blk_start, cl < blk_start + BLK))
    def _():
        def body(r, carry):
            dest = cl + r - blk_start

            @pl.when(jnp.logical_and(dest >= 0, dest < BLK))
            def _():
                out_k[pl.ds(dest, 1)] = new_keys[pl.ds(r, 1)]
                out_v[pl.ds(dest, 1)] = new_values[pl.ds(r, 1)]

            return carry

        jax.lax.fori_loop(0, NEW, body, 0)


def kernel(keys, values, current_length, new_keys, new_values):
    cl1 = current_length.reshape(1)
    out_k, out_v = pl.pallas_call(
        _extend_body,
        grid=(NBLK,),
        in_specs=[
            pl.BlockSpec(memory_space=pltpu.SMEM),
            pl.BlockSpec((BLK, G, HD), lambda i: (i, 0, 0)),
            pl.BlockSpec((BLK, G, HD), lambda i: (i, 0, 0)),
            pl.BlockSpec((NEW, G, HD), lambda i: (0, 0, 0)),
            pl.BlockSpec((NEW, G, HD), lambda i: (0, 0, 0)),
        ],
        out_specs=[
            pl.BlockSpec((BLK, G, HD), lambda i: (i, 0, 0)),
            pl.BlockSpec((BLK, G, HD), lambda i: (i, 0, 0)),
        ],
        out_shape=[
            jax.ShapeDtypeStruct((CAP, G, HD), jnp.float32),
            jax.ShapeDtypeStruct((CAP, G, HD), jnp.float32),
        ],
        compiler_params=pltpu.CompilerParams(
            dimension_semantics=("arbitrary",),
        ),
    )(cl1, keys, values, new_keys, new_values)
    return (out_k, out_v, current_length + NEW)
